# baseline (speedup 1.0000x reference)
import functools

import jax
import jax.numpy as jnp
from jax import lax
from jax.experimental import pallas as pl
from jax.experimental.pallas import tpu as pltpu
from jax.experimental.pallas import tpu_sc as plsc

N = 10000
E = 160000
B = 50
NPG = 200
S = 40
T = 2000
D = 16
ED = 4
A = 6
PD = 3

NC = 2
NS = 16
NW = NC * NS
CW = 125


def _mesh():
    return plsc.VectorSubcoreMesh(
        core_axis_name="c", subcore_axis_name="s", num_cores=NC, num_subcores=NS)


def _sc_gather(table, idx3):
    _, nch, cw = idx3.shape
    nmega = 1
    per = nch
    while per * cw * D * 4 > 320 * 1024:
        nmega *= 2
        per = nch // nmega
    assert per * nmega == nch

    @functools.partial(
        pl.kernel,
        out_type=jax.ShapeDtypeStruct((NW, nmega, per, cw, D), jnp.float32),
        mesh=_mesh(),
        scratch_types=[
            pltpu.VMEM((nch, cw), jnp.int32),
            pltpu.VMEM((per, cw, D), jnp.float32),
            pltpu.SemaphoreType.DMA,
        ],
        compiler_params=pltpu.CompilerParams(use_tc_tiling_on_sc=False),
    )
    def k(table_hbm, idx_hbm, out_hbm, idx_v, rows_v, sem):
        wid = lax.axis_index("s") * NC + lax.axis_index("c")
        pltpu.sync_copy(idx_hbm.at[wid], idx_v)
        for m in range(nmega):
            def body(j, carry, m=m):
                pltpu.async_copy(
                    table_hbm.at[idx_v.at[m * per + j]], rows_v.at[j], sem
                ).wait()
                return carry
            lax.fori_loop(0, per, body, 0)
            pltpu.sync_copy(rows_v, out_hbm.at[wid, m])

    return k(table, idx3)


def _sc_scatter(msg4, dst3, nseg):
    _, nch, cw = dst3.shape
    rows_per = nseg // NS
    nz = rows_per // cw
    assert nz * cw == rows_per
    mb = 10
    while nch % mb:
        mb -= 1
    nmega = nch // mb

    @functools.partial(
        pl.kernel,
        out_type=jax.ShapeDtypeStruct((NC, nseg, D), jnp.float32),
        mesh=_mesh(),
        scratch_types=[
            pltpu.VMEM((nch, cw), jnp.int32),
            pltpu.VMEM((mb, cw, D), jnp.float32),
            pltpu.VMEM((cw, D), jnp.float32),
            pltpu.VMEM_SHARED((nseg, D), jnp.float32),
        ],
        compiler_params=pltpu.CompilerParams(use_tc_tiling_on_sc=False),
    )
    def k(msg_hbm, dst_hbm, out_hbm, idx_v, mbuf, zbuf, agg_sh):
        c = lax.axis_index("c")
        s = lax.axis_index("s")
        wid = s * NC + c

        def zrow(i, carry):
            zbuf[i, :] = jnp.zeros((D,), jnp.float32)
            return carry
        lax.fori_loop(0, cw, zrow, 0)

        def zseg(j, carry):
            pltpu.sync_copy(zbuf, agg_sh.at[pl.ds(s * rows_per + j * cw, cw)])
            return carry
        lax.fori_loop(0, nz, zseg, 0)
        pltpu.sync_copy(dst_hbm.at[wid], idx_v)
        plsc.subcore_barrier()

        def mega(m, carry):
            pltpu.sync_copy(msg_hbm.at[wid].at[pl.ds(m * mb, mb)], mbuf)
            def one(i, carry2):
                pltpu.sync_copy(mbuf.at[i], agg_sh.at[idx_v.at[m * mb + i]],
                                add=True)
                return carry2
            lax.fori_loop(0, mb, one, 0)
            return carry
        lax.fori_loop(0, nmega, mega, 0)
        plsc.subcore_barrier()

        def wb(j, carry):
            pltpu.sync_copy(agg_sh.at[pl.ds(s * rows_per + j * cw, cw)], zbuf)
            pltpu.sync_copy(zbuf, out_hbm.at[c].at[pl.ds(s * rows_per + j * cw, cw)])
            return carry
        lax.fori_loop(0, nz, wb, 0)

    return k(msg4, dst3)


def _init_kernel(x, lin0Wt2, lin0b2):
    BN = 2000
    G = 2 * N // BN
    H = N // BN

    def body(x_ref, w_ref, b_ref, o_ref):
        o_ref[...] = jax.nn.relu(
            jnp.dot(x_ref[...], w_ref[0], preferred_element_type=jnp.float32)
            + b_ref[0])

    return pl.pallas_call(
        body,
        grid=(G,),
        in_specs=[
            pl.BlockSpec((BN, PD), lambda j: (j % H, 0)),
            pl.BlockSpec((1, PD, D), lambda j: (j // H, 0, 0)),
            pl.BlockSpec((1, 1, D), lambda j: (j // H, 0, 0)),
        ],
        out_specs=pl.BlockSpec((BN, D), lambda j: (j, 0)),
        out_shape=jax.ShapeDtypeStruct((2 * N, D), jnp.float32),
    )(x, lin0Wt2, lin0b2)


def _edge_w_kernel(edge_attr, e1Wt2, e1b2, e2Wt2, e2b2):
    BE = 4000
    G = 2 * E // BE
    H = E // BE

    def body(ea_ref, w1_ref, b1_ref, w2_ref, b2_ref, o_ref):
        he = jax.nn.relu(
            jnp.dot(ea_ref[...], w1_ref[0], preferred_element_type=jnp.float32)
            + b1_ref[0])
        o_ref[...] = (
            jnp.dot(he, w2_ref[0], preferred_element_type=jnp.float32)
            + b2_ref[0])

    return pl.pallas_call(
        body,
        grid=(G,),
        in_specs=[
            pl.BlockSpec((BE, ED), lambda j: (j % H, 0)),
            pl.BlockSpec((1, ED, D), lambda j: (j // H, 0, 0)),
            pl.BlockSpec((1, 1, D), lambda j: (j // H, 0, 0)),
            pl.BlockSpec((1, D, D * D), lambda j: (j // H, 0, 0)),
            pl.BlockSpec((1, 1, D * D), lambda j: (j // H, 0, 0)),
        ],
        out_specs=pl.BlockSpec((BE, D * D), lambda j: (j, 0)),
        out_shape=jax.ShapeDtypeStruct((2 * E, D * D), jnp.float32),
    )(edge_attr, e1Wt2, e1b2, e2Wt2, e2b2)


def _msg_kernel(xj, W_edge):
    BE = 4000
    G = 2 * E // BE

    def body(xj_ref, w_ref, o_ref):
        x = xj_ref[...]
        w = w_ref[...]
        acc = x[:, 0:1] * w[:, 0:D]
        for i in range(1, D):
            acc = acc + x[:, i:i + 1] * w[:, i * D:(i + 1) * D]
        o_ref[...] = acc

    return pl.pallas_call(
        body,
        grid=(G,),
        in_specs=[
            pl.BlockSpec((BE, D), lambda j: (j, 0)),
            pl.BlockSpec((BE, D * D), lambda j: (j, 0)),
        ],
        out_specs=pl.BlockSpec((BE, D), lambda j: (j, 0)),
        out_shape=jax.ShapeDtypeStruct((2 * E, D), jnp.float32),
    )(xj, W_edge)


def _node_kernel(aggP, cntP, table, root2, convb2, gWiht2, gWhht2, gbih2, gbhh2):
    BN = 2000
    G = 2 * N // BN
    H = N // BN

    def body(a0, a1, c0, c1, t_ref, root, cb, wih, whh, bih, bhh, o_ref):
        cnt = jnp.maximum(c0[...] + c1[...], 1.0)
        agg = (a0[...] + a1[...]) / cnt
        out = t_ref[...]
        m = jax.nn.relu(
            agg + jnp.dot(out, root[0], preferred_element_type=jnp.float32)
            + cb[0])
        gi = jnp.dot(m, wih[0], preferred_element_type=jnp.float32) + bih[0]
        gh = jnp.dot(out, whh[0], preferred_element_type=jnp.float32) + bhh[0]
        r = jax.nn.sigmoid(gi[:, :D] + gh[:, :D])
        z = jax.nn.sigmoid(gi[:, D:2 * D] + gh[:, D:2 * D])
        ng = jnp.tanh(gi[:, 2 * D:] + r * gh[:, 2 * D:])
        o_ref[...] = (1.0 - z) * ng + z * out

    a0 = aggP[0]
    a1 = aggP[1]
    c0 = cntP[0]
    c1 = cntP[1]
    return pl.pallas_call(
        body,
        grid=(G,),
        in_specs=[
            pl.BlockSpec((BN, D), lambda j: (j, 0)),
            pl.BlockSpec((BN, D), lambda j: (j, 0)),
            pl.BlockSpec((BN, D), lambda j: (j % H, 0)),
            pl.BlockSpec((BN, D), lambda j: (j % H, 0)),
            pl.BlockSpec((BN, D), lambda j: (j, 0)),
            pl.BlockSpec((1, D, D), lambda j: (j // H, 0, 0)),
            pl.BlockSpec((1, 1, D), lambda j: (j // H, 0, 0)),
            pl.BlockSpec((1, D, 3 * D), lambda j: (j // H, 0, 0)),
            pl.BlockSpec((1, D, 3 * D), lambda j: (j // H, 0, 0)),
            pl.BlockSpec((1, 1, 3 * D), lambda j: (j // H, 0, 0)),
            pl.BlockSpec((1, 1, 3 * D), lambda j: (j // H, 0, 0)),
        ],
        out_specs=pl.BlockSpec((BN, D), lambda j: (j, 0)),
        out_shape=jax.ShapeDtypeStruct((2 * N, D), jnp.float32),
    )(a0, a1, c0, c1, table, root2, convb2, gWiht2, gWhht2, gbih2, gbhh2)


def _set2set_kernel(X3, s2sWiht, s2sWhht, s2sb, memWiht, memb):
    def body(x_ref, wih_ref, whh_ref, b_ref, mwih_ref, mb_ref, hx_ref, cx_ref):
        X = x_ref[...]
        wih = wih_ref[...]
        whh = whh_ref[...]
        bb = b_ref[...]
        h = jnp.zeros((B, D), jnp.float32)
        c = jnp.zeros((B, D), jnp.float32)
        q_star = jnp.zeros((B, 2 * D), jnp.float32)
        for _ in range(6):
            g = (jnp.dot(q_star, wih, preferred_element_type=jnp.float32)
                 + jnp.dot(h, whh, preferred_element_type=jnp.float32) + bb)
            ig = jax.nn.sigmoid(g[:, :D])
            fg = jax.nn.sigmoid(g[:, D:2 * D])
            gg = jnp.tanh(g[:, 2 * D:3 * D])
            og = jax.nn.sigmoid(g[:, 3 * D:])
            c = fg * c + ig * gg
            h = og * jnp.tanh(c)
            e = jnp.sum(X * h[:, None, :], axis=-1)
            emax = jnp.max(e, axis=1, keepdims=True)
            a = jnp.exp(e - emax)
            den = jnp.sum(a, axis=1, keepdims=True)
            a = a / den
            r = jnp.sum(a[:, :, None] * X, axis=1)
            q_star = jnp.concatenate([h, r], axis=-1)
        g = jnp.dot(q_star, mwih_ref[...], preferred_element_type=jnp.float32) \
            + mb_ref[...]
        ig = jax.nn.sigmoid(g[:, :D])
        fg = jax.nn.sigmoid(g[:, D:2 * D])
        gg = jnp.tanh(g[:, 2 * D:3 * D])
        og = jax.nn.sigmoid(g[:, 3 * D:])
        c2 = ig * gg
        hx_ref[...] = og * jnp.tanh(c2)
        cx_ref[...] = c2

    return pl.pallas_call(
        body,
        out_shape=[
            jax.ShapeDtypeStruct((B, D), jnp.float32),
            jax.ShapeDtypeStruct((B, D), jnp.float32),
        ],
    )(X3, s2sWiht, s2sWhht, s2sb, memWiht, memb)


def _head_kernel(feat, l1Wt, l1b, l2Wt, l2b, gum, hv, c1Wt, c1b, c3Wt, c3b):
    def body(f_ref, w1, b1, w2, b2, g_ref, hv_ref, cw1, cb1, cw3, cb3,
             act_ref, lp_ref, ent_ref, v_ref):
        o = jax.nn.relu(
            jnp.dot(f_ref[...], w1[...], preferred_element_type=jnp.float32)
            + b1[...])
        logits = jnp.dot(o, w2[...], preferred_element_type=jnp.float32) + b2[...]
        lmax = jnp.max(logits, axis=-1, keepdims=True)
        ls = logits - lmax
        lse = jnp.log(jnp.sum(jnp.exp(ls), axis=-1, keepdims=True))
        logp = ls - lse
        z = logits + g_ref[...]
        zmax = jnp.max(z, axis=-1, keepdims=True)
        ii = lax.broadcasted_iota(jnp.int32, z.shape, 1)
        act = jnp.min(jnp.where(z >= zmax, ii, A), axis=-1, keepdims=True)
        act_ref[...] = act
        onehot = (ii == act).astype(jnp.float32)
        lp_ref[...] = jnp.sum(logp * onehot, axis=-1, keepdims=True)
        p = jnp.exp(logp)
        ent_ref[...] = -jnp.sum(p * logp, axis=-1, keepdims=True)
        vh = jax.nn.relu(
            jnp.dot(hv_ref[...], cw1[...], preferred_element_type=jnp.float32)
            + cb1[...])
        v_ref[...] = jnp.dot(vh, cw3[...], preferred_element_type=jnp.float32) \
            + cb3[...]

    return pl.pallas_call(
        body,
        out_shape=[
            jax.ShapeDtypeStruct((T, 1), jnp.int32),
            jax.ShapeDtypeStruct((T, 1), jnp.float32),
            jax.ShapeDtypeStruct((T, 1), jnp.float32),
            jax.ShapeDtypeStruct((B, 1), jnp.float32),
        ],
    )(feat, l1Wt, l1b, l2Wt, l2b, gum, hv, c1Wt, c1b, c3Wt, c3b)


def _stack2(p, name):
    return jnp.stack([p['a_' + name], p['c_' + name]])


def kernel(x, edge_index, edge_attr, batch, nonring, nrbidx, params):
    p = params
    src = edge_index[0]
    dst = edge_index[1]

    src2 = jnp.concatenate([src, src + N]).reshape(NW, 80, CW)
    dst2 = jnp.concatenate([dst, dst + N]).reshape(NW, 80, CW)
    dst1 = dst.reshape(NW, 40, CW)

    lin0Wt2 = jnp.stack([p['a_lin0_W'].T, p['c_lin0_W'].T])
    lin0b2 = jnp.stack([p['a_lin0_b'][None], p['c_lin0_b'][None]])
    e1Wt2 = jnp.stack([p['a_e1_W'].T, p['c_e1_W'].T])
    e1b2 = jnp.stack([p['a_e1_b'][None], p['c_e1_b'][None]])
    e2Wt2 = jnp.stack([p['a_e2_W'].T, p['c_e2_W'].T])
    e2b2 = jnp.stack([p['a_e2_b'][None], p['c_e2_b'][None]])
    root2 = _stack2(p, 'root')
    convb2 = jnp.stack([p['a_conv_b'][None], p['c_conv_b'][None]])
    gWiht2 = jnp.stack([p['a_gru_Wih'].T, p['c_gru_Wih'].T])
    gWhht2 = jnp.stack([p['a_gru_Whh'].T, p['c_gru_Whh'].T])
    gbih2 = jnp.stack([p['a_gru_bih'][None], p['c_gru_bih'][None]])
    gbhh2 = jnp.stack([p['a_gru_bhh'][None], p['c_gru_bhh'][None]])

    table = _init_kernel(x, lin0Wt2, lin0b2)
    W_edge = _edge_w_kernel(edge_attr, e1Wt2, e1b2, e2Wt2, e2b2)

    ones4 = jnp.ones((NW, 40, CW, D), jnp.float32)
    cntP = _sc_scatter(ones4, dst1, N)

    for _ in range(6):
        rows = _sc_gather(table, src2)
        xj = rows.reshape(2 * E, D)
        msg = _msg_kernel(xj, W_edge)
        aggP = _sc_scatter(msg.reshape(NW, 80, CW, D), dst2, 2 * N)
        table = _node_kernel(aggP, cntP, table, root2, convb2,
                             gWiht2, gWhht2, gbih2, gbhh2)

    out_a = table[:N]
    out_c = table[N:]

    hp, cp = _set2set_kernel(
        out_a.reshape(B, NPG, D),
        p['a_s2s_Wih'].T, p['a_s2s_Whh'].T,
        (p['a_s2s_bih'] + p['a_s2s_bhh'])[None],
        p['a_mem_Wih'].T,
        (p['a_mem_bih'] + p['a_mem_bhh'])[None])
    hv, cv = _set2set_kernel(
        out_c.reshape(B, NPG, D),
        p['c_s2s_Wih'].T, p['c_s2s_Whh'].T,
        (p['c_s2s_bih'] + p['c_s2s_bhh'])[None],
        p['c_mem_Wih'].T,
        (p['c_mem_bih'] + p['c_mem_bhh'])[None])

    nr_flat = nonring.reshape(-1)
    nr_idx = jnp.zeros((8192,), jnp.int32).at[:4 * T].set(nr_flat)
    nr_rows = _sc_gather(out_a, nr_idx.reshape(NW, 2, 128))
    osel = nr_rows.reshape(8192, D)[:4 * T].reshape(4, T, D)
    lsel = jnp.repeat(hp, S, axis=0)
    cat = jnp.concatenate([lsel[None], osel], axis=0)
    feat = jnp.transpose(cat, (2, 1, 0)).reshape(-1, 5 * D)

    gum = jax.random.gumbel(jax.random.key(123), (B, S, A), jnp.float32)
    act, lp, ent, v = _head_kernel(
        feat, p['a_lin1_W'].T, p['a_lin1_b'][None], p['a_lin2_W'].T,
        p['a_lin2_b'][None], gum.reshape(T, A), hv, p['c_lin1_W'].T,
        p['c_lin1_b'][None], p['c_lin3_W'].T, p['c_lin3_b'][None])

    action = act.reshape(B, S)
    lp = lp.reshape(1, B, S)
    ent = ent.reshape(1, B, S)
    v = v.reshape(1, B, 1)
    return (action, lp, ent, v, hp, cp, hv, cv)


# speedup vs baseline: 2.8826x; 2.8826x over previous
import functools

import jax
import jax.numpy as jnp
from jax import lax
from jax.experimental import pallas as pl
from jax.experimental.pallas import tpu as pltpu
from jax.experimental.pallas import tpu_sc as plsc

N = 10000
E = 160000
B = 50
NPG = 200
S = 40
T = 2000
D = 16
ED = 4
A = 6
PD = 3

NC = 2
NS = 16
NW = NC * NS
CW = 125


def _mesh():
    return plsc.VectorSubcoreMesh(
        core_axis_name="c", subcore_axis_name="s", num_cores=NC, num_subcores=NS)


def _sc_gather(table, idx3):
    _, nch, cw = idx3.shape
    nmega = 1
    per = nch
    while per * cw * D * 4 > 320 * 1024:
        nmega *= 2
        per = nch // nmega
    assert per * nmega == nch

    @functools.partial(
        pl.kernel,
        out_type=jax.ShapeDtypeStruct((NW, nmega, per, cw, D), jnp.float32),
        mesh=_mesh(),
        scratch_types=[
            pltpu.VMEM((nch, cw), jnp.int32),
            pltpu.VMEM((per, cw, D), jnp.float32),
            pltpu.SemaphoreType.DMA,
        ],
        compiler_params=pltpu.CompilerParams(use_tc_tiling_on_sc=False),
    )
    def k(table_hbm, idx_hbm, out_hbm, idx_v, rows_v, sem):
        wid = lax.axis_index("s") * NC + lax.axis_index("c")
        pltpu.sync_copy(idx_hbm.at[wid], idx_v)
        for m in range(nmega):
            def body(j, carry, m=m):
                pltpu.async_copy(
                    table_hbm.at[idx_v.at[m * per + j]], rows_v.at[j], sem
                ).wait()
                return carry
            lax.fori_loop(0, per, body, 0)
            pltpu.sync_copy(rows_v, out_hbm.at[wid, m])

    return k(table, idx3)


def _sc_scatter(msg4, dst3, nseg):
    _, nch, cw = dst3.shape
    rows_per = nseg // NS
    nz = rows_per // cw
    assert nz * cw == rows_per
    mb = 10
    while nch % mb:
        mb -= 1
    nmega = nch // mb

    @functools.partial(
        pl.kernel,
        out_type=jax.ShapeDtypeStruct((NC, nseg, D), jnp.float32),
        mesh=_mesh(),
        scratch_types=[
            pltpu.VMEM((nch, cw), jnp.int32),
            pltpu.VMEM((mb, cw, D), jnp.float32),
            pltpu.VMEM((cw, D), jnp.float32),
            pltpu.VMEM_SHARED((nseg, D), jnp.float32),
        ],
        compiler_params=pltpu.CompilerParams(use_tc_tiling_on_sc=False),
    )
    def k(msg_hbm, dst_hbm, out_hbm, idx_v, mbuf, zbuf, agg_sh):
        c = lax.axis_index("c")
        s = lax.axis_index("s")
        wid = s * NC + c

        def zrow(i, carry):
            zbuf[i, :] = jnp.zeros((D,), jnp.float32)
            return carry
        lax.fori_loop(0, cw, zrow, 0)

        def zseg(j, carry):
            pltpu.sync_copy(zbuf, agg_sh.at[pl.ds(s * rows_per + j * cw, cw)])
            return carry
        lax.fori_loop(0, nz, zseg, 0)
        pltpu.sync_copy(dst_hbm.at[wid], idx_v)
        plsc.subcore_barrier()

        def mega(m, carry):
            pltpu.sync_copy(msg_hbm.at[wid].at[pl.ds(m * mb, mb)], mbuf)
            def one(i, carry2):
                pltpu.sync_copy(mbuf.at[i], agg_sh.at[idx_v.at[m * mb + i]],
                                add=True)
                return carry2
            lax.fori_loop(0, mb, one, 0)
            return carry
        lax.fori_loop(0, nmega, mega, 0)
        plsc.subcore_barrier()

        def wb(j, carry):
            pltpu.sync_copy(agg_sh.at[pl.ds(s * rows_per + j * cw, cw)], zbuf)
            pltpu.sync_copy(zbuf, out_hbm.at[c].at[pl.ds(s * rows_per + j * cw, cw)])
            return carry
        lax.fori_loop(0, nz, wb, 0)

    return k(msg4, dst3)


def _init_kernel(x, lin0Wt2, lin0b2):
    BN = 2000
    G = 2 * N // BN
    H = N // BN

    def body(x_ref, w_ref, b_ref, o_ref):
        o_ref[...] = jax.nn.relu(
            jnp.dot(x_ref[...], w_ref[0], preferred_element_type=jnp.float32)
            + b_ref[0])

    return pl.pallas_call(
        body,
        grid=(G,),
        in_specs=[
            pl.BlockSpec((BN, PD), lambda j: (j % H, 0)),
            pl.BlockSpec((1, PD, D), lambda j: (j // H, 0, 0)),
            pl.BlockSpec((1, 1, D), lambda j: (j // H, 0, 0)),
        ],
        out_specs=pl.BlockSpec((BN, D), lambda j: (j, 0)),
        out_shape=jax.ShapeDtypeStruct((2 * N, D), jnp.float32),
    )(x, lin0Wt2, lin0b2)


def _he_kernel(edge_attr, e1Wt2, e1b2):
    BE = 4000
    G = 2 * E // BE
    H = E // BE

    def body(ea_ref, w1_ref, b1_ref, o_ref):
        o_ref[...] = jax.nn.relu(
            jnp.dot(ea_ref[...], w1_ref[0], preferred_element_type=jnp.float32)
            + b1_ref[0])

    return pl.pallas_call(
        body,
        grid=(G,),
        in_specs=[
            pl.BlockSpec((BE, ED), lambda j: (j % H, 0)),
            pl.BlockSpec((1, ED, D), lambda j: (j // H, 0, 0)),
            pl.BlockSpec((1, 1, D), lambda j: (j // H, 0, 0)),
        ],
        out_specs=pl.BlockSpec((BE, D), lambda j: (j, 0)),
        out_shape=jax.ShapeDtypeStruct((2 * E, D), jnp.float32),
    )(edge_attr, e1Wt2, e1b2)


def _msg_kernel(xj, he, P, Q, G2, Bias2):
    BE = 4000
    GR = 2 * E // BE
    H = E // BE

    def body(xj_ref, he_ref, p_ref, q_ref, g_ref, b_ref, o_ref):
        x = xj_ref[...]
        a = jnp.dot(x, p_ref[...], preferred_element_type=jnp.float32)
        bt = jnp.dot(he_ref[...], q_ref[...], preferred_element_type=jnp.float32)
        op = a * bt
        o_ref[...] = (
            jnp.dot(op, g_ref[0], preferred_element_type=jnp.float32)
            + jnp.dot(x, b_ref[0], preferred_element_type=jnp.float32))

    return pl.pallas_call(
        body,
        grid=(GR,),
        in_specs=[
            pl.BlockSpec((BE, D), lambda j: (j, 0)),
            pl.BlockSpec((BE, D), lambda j: (j, 0)),
            pl.BlockSpec((D, D * D), lambda j: (0, 0)),
            pl.BlockSpec((D, D * D), lambda j: (0, 0)),
            pl.BlockSpec((1, D * D, D), lambda j: (j // H, 0, 0)),
            pl.BlockSpec((1, D, D), lambda j: (j // H, 0, 0)),
        ],
        out_specs=pl.BlockSpec((BE, D), lambda j: (j, 0)),
        out_shape=jax.ShapeDtypeStruct((2 * E, D), jnp.float32),
    )(xj, he, P, Q, G2, Bias2)


def _node_kernel(aggP, cntP, table, root2, convb2, gWiht2, gWhht2, gbih2, gbhh2):
    BN = 2000
    G = 2 * N // BN
    H = N // BN

    def body(a0, a1, c0, c1, t_ref, root, cb, wih, whh, bih, bhh, o_ref):
        cnt = jnp.maximum(c0[...] + c1[...], 1.0)
        agg = (a0[...] + a1[...]) / cnt
        out = t_ref[...]
        m = jax.nn.relu(
            agg + jnp.dot(out, root[0], preferred_element_type=jnp.float32)
            + cb[0])
        gi = jnp.dot(m, wih[0], preferred_element_type=jnp.float32) + bih[0]
        gh = jnp.dot(out, whh[0], preferred_element_type=jnp.float32) + bhh[0]
        r = jax.nn.sigmoid(gi[:, :D] + gh[:, :D])
        z = jax.nn.sigmoid(gi[:, D:2 * D] + gh[:, D:2 * D])
        ng = jnp.tanh(gi[:, 2 * D:] + r * gh[:, 2 * D:])
        o_ref[...] = (1.0 - z) * ng + z * out

    a0 = aggP[0]
    a1 = aggP[1]
    c0 = cntP[0]
    c1 = cntP[1]
    return pl.pallas_call(
        body,
        grid=(G,),
        in_specs=[
            pl.BlockSpec((BN, D), lambda j: (j, 0)),
            pl.BlockSpec((BN, D), lambda j: (j, 0)),
            pl.BlockSpec((BN, D), lambda j: (j % H, 0)),
            pl.BlockSpec((BN, D), lambda j: (j % H, 0)),
            pl.BlockSpec((BN, D), lambda j: (j, 0)),
            pl.BlockSpec((1, D, D), lambda j: (j // H, 0, 0)),
            pl.BlockSpec((1, 1, D), lambda j: (j // H, 0, 0)),
            pl.BlockSpec((1, D, 3 * D), lambda j: (j // H, 0, 0)),
            pl.BlockSpec((1, D, 3 * D), lambda j: (j // H, 0, 0)),
            pl.BlockSpec((1, 1, 3 * D), lambda j: (j // H, 0, 0)),
            pl.BlockSpec((1, 1, 3 * D), lambda j: (j // H, 0, 0)),
        ],
        out_specs=pl.BlockSpec((BN, D), lambda j: (j, 0)),
        out_shape=jax.ShapeDtypeStruct((2 * N, D), jnp.float32),
    )(a0, a1, c0, c1, table, root2, convb2, gWiht2, gWhht2, gbih2, gbhh2)


def _set2set_kernel(X3, s2sWiht, s2sWhht, s2sb, memWiht, memb):
    def body(x_ref, wih_ref, whh_ref, b_ref, mwih_ref, mb_ref, hx_ref, cx_ref):
        X = x_ref[...]
        wih = wih_ref[...]
        whh = whh_ref[...]
        bb = b_ref[...]
        h = jnp.zeros((B, D), jnp.float32)
        c = jnp.zeros((B, D), jnp.float32)
        q_star = jnp.zeros((B, 2 * D), jnp.float32)
        for _ in range(6):
            g = (jnp.dot(q_star, wih, preferred_element_type=jnp.float32)
                 + jnp.dot(h, whh, preferred_element_type=jnp.float32) + bb)
            ig = jax.nn.sigmoid(g[:, :D])
            fg = jax.nn.sigmoid(g[:, D:2 * D])
            gg = jnp.tanh(g[:, 2 * D:3 * D])
            og = jax.nn.sigmoid(g[:, 3 * D:])
            c = fg * c + ig * gg
            h = og * jnp.tanh(c)
            e = jnp.sum(X * h[:, None, :], axis=-1)
            emax = jnp.max(e, axis=1, keepdims=True)
            a = jnp.exp(e - emax)
            den = jnp.sum(a, axis=1, keepdims=True)
            a = a / den
            r = jnp.sum(a[:, :, None] * X, axis=1)
            q_star = jnp.concatenate([h, r], axis=-1)
        g = jnp.dot(q_star, mwih_ref[...], preferred_element_type=jnp.float32) \
            + mb_ref[...]
        ig = jax.nn.sigmoid(g[:, :D])
        fg = jax.nn.sigmoid(g[:, D:2 * D])
        gg = jnp.tanh(g[:, 2 * D:3 * D])
        og = jax.nn.sigmoid(g[:, 3 * D:])
        c2 = ig * gg
        hx_ref[...] = og * jnp.tanh(c2)
        cx_ref[...] = c2

    return pl.pallas_call(
        body,
        out_shape=[
            jax.ShapeDtypeStruct((B, D), jnp.float32),
            jax.ShapeDtypeStruct((B, D), jnp.float32),
        ],
    )(X3, s2sWiht, s2sWhht, s2sb, memWiht, memb)


def _head_kernel(feat, l1Wt, l1b, l2Wt, l2b, gum, hv, c1Wt, c1b, c3Wt, c3b):
    def body(f_ref, w1, b1, w2, b2, g_ref, hv_ref, cw1, cb1, cw3, cb3,
             act_ref, lp_ref, ent_ref, v_ref):
        o = jax.nn.relu(
            jnp.dot(f_ref[...], w1[...], preferred_element_type=jnp.float32)
            + b1[...])
        logits = jnp.dot(o, w2[...], preferred_element_type=jnp.float32) + b2[...]
        lmax = jnp.max(logits, axis=-1, keepdims=True)
        ls = logits - lmax
        lse = jnp.log(jnp.sum(jnp.exp(ls), axis=-1, keepdims=True))
        logp = ls - lse
        z = logits + g_ref[...]
        zmax = jnp.max(z, axis=-1, keepdims=True)
        ii = lax.broadcasted_iota(jnp.int32, z.shape, 1)
        act = jnp.min(jnp.where(z >= zmax, ii, A), axis=-1, keepdims=True)
        act_ref[...] = act
        onehot = (ii == act).astype(jnp.float32)
        lp_ref[...] = jnp.sum(logp * onehot, axis=-1, keepdims=True)
        p = jnp.exp(logp)
        ent_ref[...] = -jnp.sum(p * logp, axis=-1, keepdims=True)
        vh = jax.nn.relu(
            jnp.dot(hv_ref[...], cw1[...], preferred_element_type=jnp.float32)
            + cb1[...])
        v_ref[...] = jnp.dot(vh, cw3[...], preferred_element_type=jnp.float32) \
            + cb3[...]

    return pl.pallas_call(
        body,
        out_shape=[
            jax.ShapeDtypeStruct((T, 1), jnp.int32),
            jax.ShapeDtypeStruct((T, 1), jnp.float32),
            jax.ShapeDtypeStruct((T, 1), jnp.float32),
            jax.ShapeDtypeStruct((B, 1), jnp.float32),
        ],
    )(feat, l1Wt, l1b, l2Wt, l2b, gum, hv, c1Wt, c1b, c3Wt, c3b)


def _stack2(p, name):
    return jnp.stack([p['a_' + name], p['c_' + name]])


def kernel(x, edge_index, edge_attr, batch, nonring, nrbidx, params):
    p = params
    src = edge_index[0]
    dst = edge_index[1]

    src2 = jnp.concatenate([src, src + N]).reshape(NW, 80, CW)
    dst2 = jnp.concatenate([dst, dst + N]).reshape(NW, 80, CW)
    dst1 = dst.reshape(NW, 40, CW)

    lin0Wt2 = jnp.stack([p['a_lin0_W'].T, p['c_lin0_W'].T])
    lin0b2 = jnp.stack([p['a_lin0_b'][None], p['c_lin0_b'][None]])
    e1Wt2 = jnp.stack([p['a_e1_W'].T, p['c_e1_W'].T])
    e1b2 = jnp.stack([p['a_e1_b'][None], p['c_e1_b'][None]])
    ii = jnp.arange(D * D, dtype=jnp.int32)
    P = (ii[None, :] // D == jnp.arange(D)[:, None]).astype(jnp.float32)
    Q = (ii[None, :] % D == jnp.arange(D)[:, None]).astype(jnp.float32)

    def _mk_g(e2W):
        w3 = e2W.reshape(D, D, D)
        return jnp.transpose(w3, (0, 2, 1)).reshape(D * D, D)

    G2 = jnp.stack([_mk_g(p['a_e2_W']), _mk_g(p['c_e2_W'])])
    Bias2 = jnp.stack([p['a_e2_b'].reshape(D, D), p['c_e2_b'].reshape(D, D)])
    root2 = _stack2(p, 'root')
    convb2 = jnp.stack([p['a_conv_b'][None], p['c_conv_b'][None]])
    gWiht2 = jnp.stack([p['a_gru_Wih'].T, p['c_gru_Wih'].T])
    gWhht2 = jnp.stack([p['a_gru_Whh'].T, p['c_gru_Whh'].T])
    gbih2 = jnp.stack([p['a_gru_bih'][None], p['c_gru_bih'][None]])
    gbhh2 = jnp.stack([p['a_gru_bhh'][None], p['c_gru_bhh'][None]])

    table = _init_kernel(x, lin0Wt2, lin0b2)
    he = _he_kernel(edge_attr, e1Wt2, e1b2)

    ones4 = jnp.ones((NW, 40, CW, D), jnp.float32)
    cntP = _sc_scatter(ones4, dst1, N)

    for _ in range(6):
        rows = _sc_gather(table, src2)
        xj = rows.reshape(2 * E, D)
        msg = _msg_kernel(xj, he, P, Q, G2, Bias2)
        aggP = _sc_scatter(msg.reshape(NW, 80, CW, D), dst2, 2 * N)
        table = _node_kernel(aggP, cntP, table, root2, convb2,
                             gWiht2, gWhht2, gbih2, gbhh2)

    out_a = table[:N]
    out_c = table[N:]

    hp, cp = _set2set_kernel(
        out_a.reshape(B, NPG, D),
        p['a_s2s_Wih'].T, p['a_s2s_Whh'].T,
        (p['a_s2s_bih'] + p['a_s2s_bhh'])[None],
        p['a_mem_Wih'].T,
        (p['a_mem_bih'] + p['a_mem_bhh'])[None])
    hv, cv = _set2set_kernel(
        out_c.reshape(B, NPG, D),
        p['c_s2s_Wih'].T, p['c_s2s_Whh'].T,
        (p['c_s2s_bih'] + p['c_s2s_bhh'])[None],
        p['c_mem_Wih'].T,
        (p['c_mem_bih'] + p['c_mem_bhh'])[None])

    nr_flat = nonring.reshape(-1)
    nr_idx = jnp.zeros((8192,), jnp.int32).at[:4 * T].set(nr_flat)
    nr_rows = _sc_gather(out_a, nr_idx.reshape(NW, 2, 128))
    osel = nr_rows.reshape(8192, D)[:4 * T].reshape(4, T, D)
    lsel = jnp.repeat(hp, S, axis=0)
    cat = jnp.concatenate([lsel[None], osel], axis=0)
    feat = jnp.transpose(cat, (2, 1, 0)).reshape(-1, 5 * D)

    gum = jax.random.gumbel(jax.random.key(123), (B, S, A), jnp.float32)
    act, lp, ent, v = _head_kernel(
        feat, p['a_lin1_W'].T, p['a_lin1_b'][None], p['a_lin2_W'].T,
        p['a_lin2_b'][None], gum.reshape(T, A), hv, p['c_lin1_W'].T,
        p['c_lin1_b'][None], p['c_lin3_W'].T, p['c_lin3_b'][None])

    action = act.reshape(B, S)
    lp = lp.reshape(1, B, S)
    ent = ent.reshape(1, B, S)
    v = v.reshape(1, B, 1)
    return (action, lp, ent, v, hp, cp, hv, cv)


# speedup vs baseline: 2.8906x; 1.0028x over previous
import functools

import jax
import jax.numpy as jnp
from jax import lax
from jax.experimental import pallas as pl
from jax.experimental.pallas import tpu as pltpu
from jax.experimental.pallas import tpu_sc as plsc

N = 10000
E = 160000
B = 50
NPG = 200
S = 40
T = 2000
D = 16
ED = 4
A = 6
PD = 3

NC = 2
NS = 16
NW = NC * NS
CW = 125


def _mesh():
    return plsc.VectorSubcoreMesh(
        core_axis_name="c", subcore_axis_name="s", num_cores=NC, num_subcores=NS)


def _sc_gather(table, idx3):
    _, nch, cw = idx3.shape
    nmega = 1
    per = nch
    while per * cw * D * 4 > 320 * 1024:
        nmega *= 2
        per = nch // nmega
    assert per * nmega == nch

    @functools.partial(
        pl.kernel,
        out_type=jax.ShapeDtypeStruct((NW, nmega, per, cw, D), jnp.float32),
        mesh=_mesh(),
        scratch_types=[
            pltpu.VMEM((nch, cw), jnp.int32),
            pltpu.VMEM((per, cw, D), jnp.float32),
            pltpu.SemaphoreType.DMA,
        ],
        compiler_params=pltpu.CompilerParams(use_tc_tiling_on_sc=False),
    )
    def k(table_hbm, idx_hbm, out_hbm, idx_v, rows_v, sem):
        wid = lax.axis_index("s") * NC + lax.axis_index("c")
        pltpu.sync_copy(idx_hbm.at[wid], idx_v)
        for m in range(nmega):
            def body(j, carry, m=m):
                pltpu.async_copy(
                    table_hbm.at[idx_v.at[m * per + j]], rows_v.at[j], sem
                ).wait()
                return carry
            lax.fori_loop(0, per, body, 0)
            pltpu.sync_copy(rows_v, out_hbm.at[wid, m])

    return k(table, idx3)


def _sc_scatter(msg4, dst3, nseg):
    _, nch, cw = dst3.shape
    rows_per = nseg // NS
    nz = rows_per // cw
    assert nz * cw == rows_per
    mb = 20
    while nch % mb:
        mb -= 1
    nmega = nch // mb

    @functools.partial(
        pl.kernel,
        out_type=jax.ShapeDtypeStruct((NC, nseg, D), jnp.float32),
        mesh=_mesh(),
        scratch_types=[
            pltpu.VMEM((nch, cw), jnp.int32),
            pltpu.VMEM((2, mb, cw, D), jnp.float32),
            pltpu.VMEM((cw, D), jnp.float32),
            pltpu.VMEM_SHARED((nseg, D), jnp.float32),
            pltpu.SemaphoreType.DMA,
            pltpu.SemaphoreType.DMA,
        ],
        compiler_params=pltpu.CompilerParams(use_tc_tiling_on_sc=False),
    )
    def k(msg_hbm, dst_hbm, out_hbm, idx_v, mbuf, zbuf, agg_sh, sl, sa):
        c = lax.axis_index("c")
        s = lax.axis_index("s")
        wid = s * NC + c

        def zrow(i, carry):
            zbuf[i, :] = jnp.zeros((D,), jnp.float32)
            return carry
        lax.fori_loop(0, cw, zrow, 0)

        def zseg(j, carry):
            pltpu.sync_copy(zbuf, agg_sh.at[pl.ds(s * rows_per + j * cw, cw)])
            return carry
        lax.fori_loop(0, nz, zseg, 0)
        pltpu.sync_copy(dst_hbm.at[wid], idx_v)
        plsc.subcore_barrier()

        def mega(m, carry):
            pltpu.sync_copy(msg_hbm.at[wid].at[pl.ds(m * mb, mb)], mbuf.at[0])
            def one(i, carry2):
                pltpu.sync_copy(mbuf.at[0].at[i], agg_sh.at[idx_v.at[m * mb + i]],
                                add=True)
                return carry2
            lax.fori_loop(0, mb, one, 0)
            return carry
        lax.fori_loop(0, nmega, mega, 0)
        plsc.subcore_barrier()

        def wb(j, carry):
            pltpu.sync_copy(agg_sh.at[pl.ds(s * rows_per + j * cw, cw)], zbuf)
            pltpu.sync_copy(zbuf, out_hbm.at[c].at[pl.ds(s * rows_per + j * cw, cw)])
            return carry
        lax.fori_loop(0, nz, wb, 0)

    return k(msg4, dst3)


def _init_kernel(x, lin0Wt2, lin0b2):
    BN = 2000
    G = 2 * N // BN
    H = N // BN

    def body(x_ref, w_ref, b_ref, o_ref):
        o_ref[...] = jax.nn.relu(
            jnp.dot(x_ref[...], w_ref[0], preferred_element_type=jnp.float32)
            + b_ref[0])

    return pl.pallas_call(
        body,
        grid=(G,),
        in_specs=[
            pl.BlockSpec((BN, PD), lambda j: (j % H, 0)),
            pl.BlockSpec((1, PD, D), lambda j: (j // H, 0, 0)),
            pl.BlockSpec((1, 1, D), lambda j: (j // H, 0, 0)),
        ],
        out_specs=pl.BlockSpec((BN, D), lambda j: (j, 0)),
        out_shape=jax.ShapeDtypeStruct((2 * N, D), jnp.float32),
    )(x, lin0Wt2, lin0b2)


def _he_kernel(edge_attr, e1Wt2, e1b2):
    BE = 4000
    G = 2 * E // BE
    H = E // BE

    def body(ea_ref, w1_ref, b1_ref, o_ref):
        o_ref[...] = jax.nn.relu(
            jnp.dot(ea_ref[...], w1_ref[0], preferred_element_type=jnp.float32)
            + b1_ref[0])

    return pl.pallas_call(
        body,
        grid=(G,),
        in_specs=[
            pl.BlockSpec((BE, ED), lambda j: (j % H, 0)),
            pl.BlockSpec((1, ED, D), lambda j: (j // H, 0, 0)),
            pl.BlockSpec((1, 1, D), lambda j: (j // H, 0, 0)),
        ],
        out_specs=pl.BlockSpec((BE, D), lambda j: (j, 0)),
        out_shape=jax.ShapeDtypeStruct((2 * E, D), jnp.float32),
    )(edge_attr, e1Wt2, e1b2)


def _msg_kernel(xj, he, P, Q, G2, Bias2):
    BE = 4000
    GR = 2 * E // BE
    H = E // BE

    def body(xj_ref, he_ref, p_ref, q_ref, g_ref, b_ref, o_ref):
        x = xj_ref[...]
        a = jnp.dot(x, p_ref[...], preferred_element_type=jnp.float32)
        bt = jnp.dot(he_ref[...], q_ref[...], preferred_element_type=jnp.float32)
        op = a * bt
        o_ref[...] = (
            jnp.dot(op, g_ref[0], preferred_element_type=jnp.float32)
            + jnp.dot(x, b_ref[0], preferred_element_type=jnp.float32))

    return pl.pallas_call(
        body,
        grid=(GR,),
        in_specs=[
            pl.BlockSpec((BE, D), lambda j: (j, 0)),
            pl.BlockSpec((BE, D), lambda j: (j, 0)),
            pl.BlockSpec((D, D * D), lambda j: (0, 0)),
            pl.BlockSpec((D, D * D), lambda j: (0, 0)),
            pl.BlockSpec((1, D * D, D), lambda j: (j // H, 0, 0)),
            pl.BlockSpec((1, D, D), lambda j: (j // H, 0, 0)),
        ],
        out_specs=pl.BlockSpec((BE, D), lambda j: (j, 0)),
        out_shape=jax.ShapeDtypeStruct((2 * E, D), jnp.float32),
    )(xj, he, P, Q, G2, Bias2)


def _node_kernel(aggP, cntP, table, root2, convb2, gWiht2, gWhht2, gbih2, gbhh2):
    BN = 2000
    G = 2 * N // BN
    H = N // BN

    def body(a0, a1, c0, c1, t_ref, root, cb, wih, whh, bih, bhh, o_ref):
        cnt = jnp.maximum(c0[...] + c1[...], 1.0)
        agg = (a0[...] + a1[...]) / cnt
        out = t_ref[...]
        m = jax.nn.relu(
            agg + jnp.dot(out, root[0], preferred_element_type=jnp.float32)
            + cb[0])
        gi = jnp.dot(m, wih[0], preferred_element_type=jnp.float32) + bih[0]
        gh = jnp.dot(out, whh[0], preferred_element_type=jnp.float32) + bhh[0]
        r = jax.nn.sigmoid(gi[:, :D] + gh[:, :D])
        z = jax.nn.sigmoid(gi[:, D:2 * D] + gh[:, D:2 * D])
        ng = jnp.tanh(gi[:, 2 * D:] + r * gh[:, 2 * D:])
        o_ref[...] = (1.0 - z) * ng + z * out

    a0 = aggP[0]
    a1 = aggP[1]
    c0 = cntP[0]
    c1 = cntP[1]
    return pl.pallas_call(
        body,
        grid=(G,),
        in_specs=[
            pl.BlockSpec((BN, D), lambda j: (j, 0)),
            pl.BlockSpec((BN, D), lambda j: (j, 0)),
            pl.BlockSpec((BN, D), lambda j: (j % H, 0)),
            pl.BlockSpec((BN, D), lambda j: (j % H, 0)),
            pl.BlockSpec((BN, D), lambda j: (j, 0)),
            pl.BlockSpec((1, D, D), lambda j: (j // H, 0, 0)),
            pl.BlockSpec((1, 1, D), lambda j: (j // H, 0, 0)),
            pl.BlockSpec((1, D, 3 * D), lambda j: (j // H, 0, 0)),
            pl.BlockSpec((1, D, 3 * D), lambda j: (j // H, 0, 0)),
            pl.BlockSpec((1, 1, 3 * D), lambda j: (j // H, 0, 0)),
            pl.BlockSpec((1, 1, 3 * D), lambda j: (j // H, 0, 0)),
        ],
        out_specs=pl.BlockSpec((BN, D), lambda j: (j, 0)),
        out_shape=jax.ShapeDtypeStruct((2 * N, D), jnp.float32),
    )(a0, a1, c0, c1, table, root2, convb2, gWiht2, gWhht2, gbih2, gbhh2)


def _set2set_kernel(X3, s2sWiht, s2sWhht, s2sb, memWiht, memb):
    def body(x_ref, wih_ref, whh_ref, b_ref, mwih_ref, mb_ref, hx_ref, cx_ref):
        X = x_ref[...]
        wih = wih_ref[...]
        whh = whh_ref[...]
        bb = b_ref[...]
        h = jnp.zeros((B, D), jnp.float32)
        c = jnp.zeros((B, D), jnp.float32)
        q_star = jnp.zeros((B, 2 * D), jnp.float32)
        for _ in range(6):
            g = (jnp.dot(q_star, wih, preferred_element_type=jnp.float32)
                 + jnp.dot(h, whh, preferred_element_type=jnp.float32) + bb)
            ig = jax.nn.sigmoid(g[:, :D])
            fg = jax.nn.sigmoid(g[:, D:2 * D])
            gg = jnp.tanh(g[:, 2 * D:3 * D])
            og = jax.nn.sigmoid(g[:, 3 * D:])
            c = fg * c + ig * gg
            h = og * jnp.tanh(c)
            e = jnp.sum(X * h[:, None, :], axis=-1)
            emax = jnp.max(e, axis=1, keepdims=True)
            a = jnp.exp(e - emax)
            den = jnp.sum(a, axis=1, keepdims=True)
            a = a / den
            r = jnp.sum(a[:, :, None] * X, axis=1)
            q_star = jnp.concatenate([h, r], axis=-1)
        g = jnp.dot(q_star, mwih_ref[...], preferred_element_type=jnp.float32) \
            + mb_ref[...]
        ig = jax.nn.sigmoid(g[:, :D])
        fg = jax.nn.sigmoid(g[:, D:2 * D])
        gg = jnp.tanh(g[:, 2 * D:3 * D])
        og = jax.nn.sigmoid(g[:, 3 * D:])
        c2 = ig * gg
        hx_ref[...] = og * jnp.tanh(c2)
        cx_ref[...] = c2

    return pl.pallas_call(
        body,
        out_shape=[
            jax.ShapeDtypeStruct((B, D), jnp.float32),
            jax.ShapeDtypeStruct((B, D), jnp.float32),
        ],
    )(X3, s2sWiht, s2sWhht, s2sb, memWiht, memb)


def _head_kernel(feat, l1Wt, l1b, l2Wt, l2b, gum, hv, c1Wt, c1b, c3Wt, c3b):
    def body(f_ref, w1, b1, w2, b2, g_ref, hv_ref, cw1, cb1, cw3, cb3,
             act_ref, lp_ref, ent_ref, v_ref):
        o = jax.nn.relu(
            jnp.dot(f_ref[...], w1[...], preferred_element_type=jnp.float32)
            + b1[...])
        logits = jnp.dot(o, w2[...], preferred_element_type=jnp.float32) + b2[...]
        lmax = jnp.max(logits, axis=-1, keepdims=True)
        ls = logits - lmax
        lse = jnp.log(jnp.sum(jnp.exp(ls), axis=-1, keepdims=True))
        logp = ls - lse
        z = logits + g_ref[...]
        zmax = jnp.max(z, axis=-1, keepdims=True)
        ii = lax.broadcasted_iota(jnp.int32, z.shape, 1)
        act = jnp.min(jnp.where(z >= zmax, ii, A), axis=-1, keepdims=True)
        act_ref[...] = act
        onehot = (ii == act).astype(jnp.float32)
        lp_ref[...] = jnp.sum(logp * onehot, axis=-1, keepdims=True)
        p = jnp.exp(logp)
        ent_ref[...] = -jnp.sum(p * logp, axis=-1, keepdims=True)
        vh = jax.nn.relu(
            jnp.dot(hv_ref[...], cw1[...], preferred_element_type=jnp.float32)
            + cb1[...])
        v_ref[...] = jnp.dot(vh, cw3[...], preferred_element_type=jnp.float32) \
            + cb3[...]

    return pl.pallas_call(
        body,
        out_shape=[
            jax.ShapeDtypeStruct((T, 1), jnp.int32),
            jax.ShapeDtypeStruct((T, 1), jnp.float32),
            jax.ShapeDtypeStruct((T, 1), jnp.float32),
            jax.ShapeDtypeStruct((B, 1), jnp.float32),
        ],
    )(feat, l1Wt, l1b, l2Wt, l2b, gum, hv, c1Wt, c1b, c3Wt, c3b)


def _stack2(p, name):
    return jnp.stack([p['a_' + name], p['c_' + name]])


def kernel(x, edge_index, edge_attr, batch, nonring, nrbidx, params):
    p = params
    src = edge_index[0]
    dst = edge_index[1]

    src2 = jnp.concatenate([src, src + N]).reshape(NW, 80, CW)
    dst2 = jnp.concatenate([dst, dst + N]).reshape(NW, 80, CW)
    dst1 = dst.reshape(NW, 40, CW)

    lin0Wt2 = jnp.stack([p['a_lin0_W'].T, p['c_lin0_W'].T])
    lin0b2 = jnp.stack([p['a_lin0_b'][None], p['c_lin0_b'][None]])
    e1Wt2 = jnp.stack([p['a_e1_W'].T, p['c_e1_W'].T])
    e1b2 = jnp.stack([p['a_e1_b'][None], p['c_e1_b'][None]])
    ii = jnp.arange(D * D, dtype=jnp.int32)
    P = (ii[None, :] // D == jnp.arange(D)[:, None]).astype(jnp.float32)
    Q = (ii[None, :] % D == jnp.arange(D)[:, None]).astype(jnp.float32)

    def _mk_g(e2W):
        w3 = e2W.reshape(D, D, D)
        return jnp.transpose(w3, (0, 2, 1)).reshape(D * D, D)

    G2 = jnp.stack([_mk_g(p['a_e2_W']), _mk_g(p['c_e2_W'])])
    Bias2 = jnp.stack([p['a_e2_b'].reshape(D, D), p['c_e2_b'].reshape(D, D)])
    root2 = _stack2(p, 'root')
    convb2 = jnp.stack([p['a_conv_b'][None], p['c_conv_b'][None]])
    gWiht2 = jnp.stack([p['a_gru_Wih'].T, p['c_gru_Wih'].T])
    gWhht2 = jnp.stack([p['a_gru_Whh'].T, p['c_gru_Whh'].T])
    gbih2 = jnp.stack([p['a_gru_bih'][None], p['c_gru_bih'][None]])
    gbhh2 = jnp.stack([p['a_gru_bhh'][None], p['c_gru_bhh'][None]])

    table = _init_kernel(x, lin0Wt2, lin0b2)
    he = _he_kernel(edge_attr, e1Wt2, e1b2)

    ones4 = jnp.ones((NW, 40, CW, D), jnp.float32)
    cntP = _sc_scatter(ones4, dst1, N)

    for _ in range(6):
        rows = _sc_gather(table, src2)
        xj = rows.reshape(2 * E, D)
        msg = _msg_kernel(xj, he, P, Q, G2, Bias2)
        aggP = _sc_scatter(msg.reshape(NW, 80, CW, D), dst2, 2 * N)
        table = _node_kernel(aggP, cntP, table, root2, convb2,
                             gWiht2, gWhht2, gbih2, gbhh2)

    out_a = table[:N]
    out_c = table[N:]

    hp, cp = _set2set_kernel(
        out_a.reshape(B, NPG, D),
        p['a_s2s_Wih'].T, p['a_s2s_Whh'].T,
        (p['a_s2s_bih'] + p['a_s2s_bhh'])[None],
        p['a_mem_Wih'].T,
        (p['a_mem_bih'] + p['a_mem_bhh'])[None])
    hv, cv = _set2set_kernel(
        out_c.reshape(B, NPG, D),
        p['c_s2s_Wih'].T, p['c_s2s_Whh'].T,
        (p['c_s2s_bih'] + p['c_s2s_bhh'])[None],
        p['c_mem_Wih'].T,
        (p['c_mem_bih'] + p['c_mem_bhh'])[None])

    nr_flat = nonring.reshape(-1)
    nr_idx = jnp.zeros((8192,), jnp.int32).at[:4 * T].set(nr_flat)
    nr_rows = _sc_gather(out_a, nr_idx.reshape(NW, 2, 128))
    osel = nr_rows.reshape(8192, D)[:4 * T].reshape(4, T, D)
    lsel = jnp.repeat(hp, S, axis=0)
    cat = jnp.concatenate([lsel[None], osel], axis=0)
    feat = jnp.transpose(cat, (2, 1, 0)).reshape(-1, 5 * D)

    gum = jax.random.gumbel(jax.random.key(123), (B, S, A), jnp.float32)
    act, lp, ent, v = _head_kernel(
        feat, p['a_lin1_W'].T, p['a_lin1_b'][None], p['a_lin2_W'].T,
        p['a_lin2_b'][None], gum.reshape(T, A), hv, p['c_lin1_W'].T,
        p['c_lin1_b'][None], p['c_lin3_W'].T, p['c_lin3_b'][None])

    action = act.reshape(B, S)
    lp = lp.reshape(1, B, S)
    ent = ent.reshape(1, B, S)
    v = v.reshape(1, B, 1)
    return (action, lp, ent, v, hp, cp, hv, cv)
